# baseline (speedup 1.0000x reference)
import jax
import jax.numpy as jnp
from jax import lax
from jax.experimental import pallas as pl
from jax.experimental.pallas import tpu as pltpu
from jax.experimental.pallas import tpu_sc as plsc

N_NODES = 10000
HIDDEN = 128
NUM_EDGE_TYPES = 4
E_PER_TYPE = 80000
TIMESTEPS = 8
N_POS = 512

NC = 2
NS = 16
HALF = N_NODES // 2
CHUNK = 128
NCHB = 11
CAPT = NCHB * CHUNK
CAPB = CAPT * NS
ACC_ROWS = 5120
TRASH = HALF + 56
TROWS = 320
TLAST = HALF - (NS - 1) * TROWS

ROW_BLK = 1000
GRID = N_NODES // ROW_BLK


def _msg_tables(h_blk, w_cat, b_cat):
    return jnp.dot(h_blk, w_cat, preferred_element_type=jnp.float32) + b_cat


def _tc_pre_body(h_ref, wcat_ref, bcat_ref, t0, t1, t2, t3):
    ht = _msg_tables(h_ref[...], wcat_ref[...], bcat_ref[...])
    t0[...] = ht[:, 0:128]
    t1[...] = ht[:, 128:256]
    t2[...] = ht[:, 256:384]
    t3[...] = ht[:, 384:512]


def _tc_gru_body(inc_ref, h_ref, wih_ref, whh_ref, bih_ref, bhh_ref,
                 wcat_ref, bcat_ref, newh_ref, t0, t1, t2, t3):
    inc = inc_ref[...]
    h = h_ref[...]
    gi = jnp.dot(inc, wih_ref[...], preferred_element_type=jnp.float32) + bih_ref[...]
    gh = jnp.dot(h, whh_ref[...], preferred_element_type=jnp.float32) + bhh_ref[...]
    r = jax.nn.sigmoid(gi[:, 0:128] + gh[:, 0:128])
    z = jax.nn.sigmoid(gi[:, 128:256] + gh[:, 128:256])
    n = jnp.tanh(gi[:, 256:384] + r * gh[:, 256:384])
    nh = (1.0 - z) * n + z * h
    newh_ref[...] = nh
    ht = _msg_tables(nh, wcat_ref[...], bcat_ref[...])
    t0[...] = ht[:, 0:128]
    t1[...] = ht[:, 128:256]
    t2[...] = ht[:, 256:384]
    t3[...] = ht[:, 384:512]


_HT_OUT = [jax.ShapeDtypeStruct((N_NODES, HIDDEN), jnp.float32)] * 4
_HT_SPECS = [pl.BlockSpec((ROW_BLK, HIDDEN), lambda i: (i, 0))] * 4
_FULL2D = lambda shape: pl.BlockSpec(shape, lambda i: (0, 0))

_tc_pre = pl.pallas_call(
    _tc_pre_body,
    grid=(GRID,),
    in_specs=[
        pl.BlockSpec((ROW_BLK, HIDDEN), lambda i: (i, 0)),
        _FULL2D((HIDDEN, 4 * HIDDEN)),
        _FULL2D((1, 4 * HIDDEN)),
    ],
    out_specs=_HT_SPECS,
    out_shape=_HT_OUT,
)

_tc_gru = pl.pallas_call(
    _tc_gru_body,
    grid=(GRID,),
    in_specs=[
        pl.BlockSpec((ROW_BLK, HIDDEN), lambda i: (i, 0)),
        pl.BlockSpec((ROW_BLK, HIDDEN), lambda i: (i, 0)),
        _FULL2D((HIDDEN, 3 * HIDDEN)),
        _FULL2D((HIDDEN, 3 * HIDDEN)),
        _FULL2D((1, 3 * HIDDEN)),
        _FULL2D((1, 3 * HIDDEN)),
        _FULL2D((HIDDEN, 4 * HIDDEN)),
        _FULL2D((1, 4 * HIDDEN)),
    ],
    out_specs=[pl.BlockSpec((ROW_BLK, HIDDEN), lambda i: (i, 0))] + _HT_SPECS,
    out_shape=[jax.ShapeDtypeStruct((N_NODES, HIDDEN), jnp.float32)] + _HT_OUT,
)


def _sc_scatter_body(t0, t1, t2, t3, srcp, dstp, zeros_hbm, out_hbm,
                     tbl, acc, src_v, dst_v, rows_v, gs0, gs1):
    c = lax.axis_index("c")
    s = lax.axis_index("s")
    gsems = (gs0, gs1)

    pltpu.sync_copy(zeros_hbm, acc.at[pl.ds(s * TROWS, TROWS)])

    for e, table in enumerate((t0, t1, t2, t3)):
        for sh in range(2):
            @pl.when(s < NS - 1)
            def _():
                pltpu.sync_copy(table.at[pl.ds(sh * HALF + s * TROWS, TROWS)],
                                tbl.at[pl.ds(s * TROWS, TROWS)])

            @pl.when(s == NS - 1)
            def _():
                pltpu.sync_copy(table.at[pl.ds(sh * HALF + s * TROWS, TLAST)],
                                tbl.at[pl.ds(s * TROWS, TLAST)])

            plsc.subcore_barrier()

            pltpu.sync_copy(srcp.at[e, c, sh, s], src_v)
            pltpu.sync_copy(dstp.at[e, c, sh, s], dst_v)

            pltpu.async_copy(tbl.at[src_v.at[0]], rows_v.at[0], gs0).wait()

            def chunk2(i, carry):
                for b in range(2):
                    j = 2 * i + b
                    cp = pltpu.async_copy(tbl.at[src_v.at[j + 1]],
                                          rows_v.at[1 - b], gsems[1 - b])
                    pltpu.sync_copy(rows_v.at[b], acc.at[dst_v.at[j]], add=True)
                    cp.wait()
                return carry

            lax.fori_loop(0, (NCHB - 1) // 2, chunk2, 0)
            pltpu.sync_copy(rows_v.at[(NCHB - 1) % 2], acc.at[dst_v.at[NCHB - 1]],
                            add=True)
            plsc.subcore_barrier()

    @pl.when(s < NS - 1)
    def _():
        pltpu.sync_copy(acc.at[pl.ds(s * TROWS, TROWS)],
                        out_hbm.at[pl.ds(c * HALF + s * TROWS, TROWS)])

    @pl.when(s == NS - 1)
    def _():
        pltpu.sync_copy(acc.at[pl.ds(s * TROWS, TLAST)],
                        out_hbm.at[pl.ds(c * HALF + s * TROWS, TLAST)])


_sc_scatter = pl.kernel(
    _sc_scatter_body,
    mesh=plsc.VectorSubcoreMesh(core_axis_name="c", subcore_axis_name="s"),
    out_type=jax.ShapeDtypeStruct((N_NODES, HIDDEN), jnp.float32),
    scratch_types=[
        pltpu.VMEM_SHARED((HALF, HIDDEN), jnp.float32),
        pltpu.VMEM_SHARED((ACC_ROWS, HIDDEN), jnp.float32),
        pltpu.VMEM((NCHB, CHUNK), jnp.int32),
        pltpu.VMEM((NCHB, CHUNK), jnp.int32),
        pltpu.VMEM((2, CHUNK, HIDDEN), jnp.float32),
        pltpu.SemaphoreType.DMA,
        pltpu.SemaphoreType.DMA,
    ],
)


def _sc_gather_body(h_hbm, pos_hbm, out_hbm, idx_v, rows_v, sem):
    wid = lax.axis_index("s") * NC + lax.axis_index("c")
    bpw = N_POS // (NC * NS)
    base = wid * bpw
    pltpu.sync_copy(pos_hbm.at[pl.ds(base, bpw)], idx_v)
    pltpu.async_copy(h_hbm.at[idx_v], rows_v, sem).wait()
    pltpu.sync_copy(rows_v, out_hbm.at[pl.ds(base, bpw)])


_sc_gather = pl.kernel(
    _sc_gather_body,
    mesh=plsc.VectorSubcoreMesh(core_axis_name="c", subcore_axis_name="s"),
    out_type=jax.ShapeDtypeStruct((N_POS, HIDDEN), jnp.float32),
    scratch_types=[
        pltpu.VMEM((N_POS // (NC * NS),), jnp.int32),
        pltpu.VMEM((N_POS // (NC * NS), HIDDEN), jnp.float32),
        pltpu.SemaphoreType.DMA,
    ],
)


def _bucket_edges(edges):
    src = edges[:, :, 0]
    dst = edges[:, :, 1]
    srcs, dsts = [], []
    for c in range(NC):
        for sh in range(2):
            m = (dst >= HALF) == (c == 1)
            m = m & ((src >= HALF) == (sh == 1))
            pos = jnp.cumsum(m.astype(jnp.int32), axis=1) - 1
            pos = jnp.where(m, pos, CAPB)
            spread = jnp.arange(CAPB, dtype=jnp.int32)
            sl = jnp.broadcast_to(spread % 4096, (NUM_EDGE_TYPES, CAPB))
            dl = jnp.broadcast_to(TRASH + (spread % 64), (NUM_EDGE_TYPES, CAPB))
            sl = sl.at[jnp.arange(NUM_EDGE_TYPES)[:, None], pos].set(
                src - sh * HALF, mode="drop")
            dl = dl.at[jnp.arange(NUM_EDGE_TYPES)[:, None], pos].set(
                dst - c * HALF, mode="drop")
            srcs.append(sl)
            dsts.append(dl)
    srcp = jnp.stack(srcs, axis=1).reshape(NUM_EDGE_TYPES, NC, 2, NS, NCHB, CHUNK)
    dstp = jnp.stack(dsts, axis=1).reshape(NUM_EDGE_TYPES, NC, 2, NS, NCHB, CHUNK)
    return srcp, dstp


def kernel(initial_node_representation, edges, node_positions, edge_W, edge_b,
           w_ih, w_hh, b_ih, b_hh):
    h = initial_node_representation
    srcp, dstp = _bucket_edges(edges)
    w_cat = jnp.concatenate([edge_W[e] for e in range(NUM_EDGE_TYPES)], axis=1)
    b_cat = edge_b.reshape(1, NUM_EDGE_TYPES * HIDDEN)
    w_ihT = w_ih.T
    w_hhT = w_hh.T
    b_ih2 = b_ih.reshape(1, 3 * HIDDEN)
    b_hh2 = b_hh.reshape(1, 3 * HIDDEN)
    zeros = jnp.zeros((TROWS, HIDDEN), jnp.float32)

    t0, t1, t2, t3 = _tc_pre(h, w_cat, b_cat)
    for _ in range(TIMESTEPS):
        inc = _sc_scatter(t0, t1, t2, t3, srcp, dstp, zeros)
        h, t0, t1, t2, t3 = _tc_gru(inc, h, w_ihT, w_hhT, b_ih2, b_hh2,
                                    w_cat, b_cat)
    return _sc_gather(h, node_positions)
